# baseline (speedup 1.0000x reference)
import functools

import jax
import jax.numpy as jnp
from jax import lax
from jax.experimental import pallas as pl
from jax.experimental.pallas import tpu as pltpu
from jax.experimental.pallas import tpu_sc as plsc

VOCAB = 100000
N_SEG = 3
MAX_LEN = 200
EMB = 64
BATCH = 4096

NC = 2
NS = 16
NW = NC * NS
NSEQ = BATCH // NW
HALF = MAX_LEN // 2
NV = EMB // 16
SLOTS = 4
OFFS = (0, 16, 32, 48, 64, 80, 84)


def _body(seq_hbm, lbl_hbm, tok_hbm, seg_hbm, pos_hbm, pidx_hbm, out_hbm,
          seg_v, pidx_v, idx_v, lbl_v, cidx_v, idx2_v, rows_v,
          posseg_sh, bsem, isems, psems, gsems, ssems):
    sid = lax.axis_index("s")
    wid = sid * NC + lax.axis_index("c")
    base = wid * NSEQ

    @pl.when(sid == 0)
    def _():
        for h in range(2):
            pltpu.sync_copy(pidx_hbm.at[h].at[pl.ds(0, HALF)], pidx_v.at[h])
        pltpu.sync_copy(seg_hbm, seg_v)
        for h in range(2):
            pltpu.async_copy(pos_hbm.at[pidx_v.at[h]],
                             rows_v.at[0].at[pl.ds(h * HALF, HALF)],
                             bsem).wait()
        for s in range(N_SEG):
            seg_vals = [seg_v[s, pl.ds(j * 16, 16)] for j in range(NV)]

            def seg_body(l, _, s=s, seg_vals=seg_vals):
                for j in range(NV):
                    rows_v[1 + s, l, pl.ds(j * 16, 16)] = (
                        rows_v[0, l, pl.ds(j * 16, 16)] + seg_vals[j])
                return 0

            lax.fori_loop(0, MAX_LEN, seg_body, 0)
            pltpu.sync_copy(rows_v.at[1 + s],
                            posseg_sh.at[pl.ds(s * MAX_LEN, MAX_LEN)])
    plsc.subcore_barrier()

    def start_idx(i, sl):
        for h in range(2):
            pltpu.async_copy(seq_hbm.at[2 * (base + i) + h],
                             idx_v.at[sl].at[h], isems[sl])
            pltpu.async_copy(lbl_hbm.at[2 * (base + i) + h],
                             lbl_v.at[sl].at[h], isems[sl])

    def wait_idx(sl):
        for h in range(2):
            pltpu.make_async_copy(seq_hbm.at[0], idx_v.at[sl].at[h],
                                  isems[sl]).wait()
            pltpu.make_async_copy(lbl_hbm.at[0], lbl_v.at[sl].at[h],
                                  isems[sl]).wait()

    def cidx_compute(sl):
        for h in range(2):
            for off in OFFS:
                lv = lax.iota(jnp.int32, 16) + (h * HALF + off)
                sv = lbl_v[sl, h, pl.ds(off, 16)]
                cidx_v[sl, h, pl.ds(off, 16)] = sv * MAX_LEN + lv
                idx2_v[sl, h, pl.ds(off, 16)] = idx_v[sl, h, pl.ds(off, 16)]

    def start_posseg(sl):
        for h in range(2):
            pltpu.async_copy(posseg_sh.at[cidx_v.at[sl].at[h]],
                             rows_v.at[sl].at[pl.ds(h * HALF, HALF)],
                             psems[sl])

    def wait_posseg(sl):
        for h in range(2):
            pltpu.make_async_copy(posseg_sh.at[cidx_v.at[sl].at[h]],
                                  rows_v.at[sl].at[pl.ds(h * HALF, HALF)],
                                  psems[sl]).wait()

    def start_tokadd(sl):
        for h in range(2):
            pltpu.async_copy(tok_hbm.at[idx2_v.at[sl].at[h]],
                             rows_v.at[sl].at[pl.ds(h * HALF, HALF)],
                             gsems[sl], add=True)

    def wait_tokadd(sl):
        for h in range(2):
            pltpu.make_async_copy(tok_hbm.at[idx2_v.at[sl].at[h]],
                                  rows_v.at[sl].at[pl.ds(h * HALF, HALF)],
                                  gsems[sl]).wait()

    def start_scatter(i, sl):
        pltpu.async_copy(rows_v.at[sl].at[:, pl.ds(0, EMB)],
                         out_hbm.at[base + i].at[:, pl.ds(0, EMB)], ssems[sl])

    def wait_scatter(sl):
        pltpu.make_async_copy(rows_v.at[sl].at[:, pl.ds(0, EMB)],
                              out_hbm.at[0].at[:, pl.ds(0, EMB)],
                              ssems[sl]).wait()

    start_idx(0, 0)
    start_idx(1, 1)
    start_idx(2, 2)
    wait_idx(0)
    cidx_compute(0)
    start_posseg(0)
    wait_idx(1)
    cidx_compute(1)
    start_posseg(1)
    wait_posseg(0)
    start_tokadd(0)

    def macro_body(m, _):
        i0 = m * SLOTS
        for u in range(SLOTS):
            i = i0 + u
            sl = u
            sl1 = (u + 1) % SLOTS
            sl2 = (u + 2) % SLOTS
            sl3 = (u + 3) % SLOTS

            @pl.when(i + 3 < NSEQ)
            def _():
                start_idx(i + 3, sl3)

            @pl.when(i + 2 < NSEQ)
            def _():
                @pl.when(i >= 2)
                def _():
                    wait_scatter(sl2)
                wait_idx(sl2)
                cidx_compute(sl2)
                start_posseg(sl2)

            @pl.when(i + 1 < NSEQ)
            def _():
                wait_posseg(sl1)
                start_tokadd(sl1)

            wait_tokadd(sl)
            start_scatter(i, sl)
        return 0

    lax.fori_loop(0, NSEQ // SLOTS, macro_body, 0)
    for sl in range(SLOTS):
        wait_scatter(sl)


def kernel(sequence, segment_label, token_table, segment_table,
           position_table, pos_inp):
    seq = jnp.pad(jnp.asarray(sequence, jnp.int32).reshape(2 * BATCH, HALF),
                  ((0, 0), (0, 128 - HALF)))
    lbl = jnp.pad(jnp.asarray(segment_label, jnp.int32).reshape(2 * BATCH, HALF),
                  ((0, 0), (0, 128 - HALF)))
    tok = jnp.pad(token_table, ((0, 0), (0, EMB)))
    seg = jnp.pad(segment_table, ((0, N_SEG + 2), (0, EMB)))
    pos = jnp.pad(position_table, ((0, 0), (0, EMB)))
    pidx = jnp.pad(jnp.asarray(pos_inp, jnp.int32).reshape(2, HALF),
                   ((0, 6), (0, 128 - HALF)))

    run = pl.kernel(
        _body,
        out_type=jax.ShapeDtypeStruct((BATCH, MAX_LEN, 2 * EMB), jnp.float32),
        mesh=plsc.VectorSubcoreMesh(core_axis_name="c", subcore_axis_name="s"),
        compiler_params=pltpu.CompilerParams(use_tc_tiling_on_sc=False),
        scratch_types=[
            pltpu.VMEM((2 * N_SEG + 2, 2 * EMB), jnp.float32),
            pltpu.VMEM((2, HALF), jnp.int32),
            pltpu.VMEM((SLOTS, 2, 128), jnp.int32),
            pltpu.VMEM((SLOTS, 2, 128), jnp.int32),
            pltpu.VMEM((SLOTS, 2, HALF), jnp.int32),
            pltpu.VMEM((SLOTS, 2, HALF), jnp.int32),
            pltpu.VMEM((SLOTS, MAX_LEN, 2 * EMB), jnp.float32),
            pltpu.VMEM_SHARED((N_SEG * MAX_LEN, 2 * EMB), jnp.float32),
            pltpu.SemaphoreType.DMA,
            [pltpu.SemaphoreType.DMA] * SLOTS,
            [pltpu.SemaphoreType.DMA] * SLOTS,
            [pltpu.SemaphoreType.DMA] * SLOTS,
            [pltpu.SemaphoreType.DMA] * SLOTS,
        ],
    )
    out = run(seq, lbl, tok, seg, pos, pidx)
    return out[:, :, :EMB]


# speedup vs baseline: 1.1921x; 1.1921x over previous
import functools

import jax
import jax.numpy as jnp
from jax import lax
from jax.experimental import pallas as pl
from jax.experimental.pallas import tpu as pltpu
from jax.experimental.pallas import tpu_sc as plsc

VOCAB = 100000
N_SEG = 3
MAX_LEN = 200
EMB = 64
BATCH = 4096

NC = 2
NS = 16
NW = NC * NS
NSEQ = BATCH // NW
HALF = MAX_LEN // 2
NV = EMB // 16
SLOTS = 4
OFFS = (0, 16, 32, 48, 64, 80, 84)


def _body(seq_hbm, lbl_hbm, tok_hbm, seg_hbm, pos_hbm, pidx_hbm, out_hbm,
          pos_v, seg_v, pidx_v, idx_v, lbl_v, cidx_v, idx2_v, rows_v,
          posseg_sh, bsem, isems, psems, gsems, ssems):
    sid = lax.axis_index("s")
    wid = sid * NC + lax.axis_index("c")
    base = wid * NSEQ

    @pl.when(sid == 0)
    def _():
        pltpu.sync_copy(pidx_hbm, pidx_v)
        pltpu.sync_copy(seg_hbm, seg_v)
        for h in range(2):
            pltpu.async_copy(pos_hbm.at[pidx_v.at[h]],
                             pos_v.at[pl.ds(h * HALF, HALF)], bsem).wait()
        for s in range(N_SEG):
            seg_vals = [seg_v[s, pl.ds(j * 16, 16)] for j in range(NV)]

            def seg_body(l, _, s=s, seg_vals=seg_vals):
                for j in range(NV):
                    rows_v[s, l, pl.ds(j * 16, 16)] = (
                        pos_v[l, pl.ds(j * 16, 16)] + seg_vals[j])
                return 0

            lax.fori_loop(0, MAX_LEN, seg_body, 0)
            pltpu.sync_copy(rows_v.at[s],
                            posseg_sh.at[pl.ds(s * MAX_LEN, MAX_LEN)])
    plsc.subcore_barrier()

    def start_idx(i, sl):
        for h in range(2):
            pltpu.async_copy(seq_hbm.at[2 * (base + i) + h],
                             idx_v.at[sl].at[h], isems[sl])
            pltpu.async_copy(lbl_hbm.at[2 * (base + i) + h],
                             lbl_v.at[sl].at[h], isems[sl])

    def wait_idx(sl):
        for h in range(2):
            pltpu.make_async_copy(seq_hbm.at[0], idx_v.at[sl].at[h],
                                  isems[sl]).wait()
            pltpu.make_async_copy(lbl_hbm.at[0], lbl_v.at[sl].at[h],
                                  isems[sl]).wait()

    def cidx_compute(sl):
        for h in range(2):
            for off in OFFS:
                lv = lax.iota(jnp.int32, 16) + (h * HALF + off)
                sv = lbl_v[sl, h, pl.ds(off, 16)]
                cidx_v[sl, h, pl.ds(off, 16)] = sv * MAX_LEN + lv
                idx2_v[sl, h, pl.ds(off, 16)] = idx_v[sl, h, pl.ds(off, 16)]

    def start_posseg(sl):
        for h in range(2):
            pltpu.async_copy(posseg_sh.at[cidx_v.at[sl].at[h]],
                             rows_v.at[sl].at[pl.ds(h * HALF, HALF)],
                             psems[sl])

    def wait_posseg(sl):
        for h in range(2):
            pltpu.make_async_copy(posseg_sh.at[cidx_v.at[sl].at[h]],
                                  rows_v.at[sl].at[pl.ds(h * HALF, HALF)],
                                  psems[sl]).wait()

    def start_tokadd(sl):
        for h in range(2):
            pltpu.async_copy(tok_hbm.at[idx2_v.at[sl].at[h]],
                             rows_v.at[sl].at[pl.ds(h * HALF, HALF)],
                             gsems[sl], add=True)

    def wait_tokadd(sl):
        for h in range(2):
            pltpu.make_async_copy(tok_hbm.at[idx2_v.at[sl].at[h]],
                                  rows_v.at[sl].at[pl.ds(h * HALF, HALF)],
                                  gsems[sl]).wait()

    def start_scatter(i, sl):
        pltpu.async_copy(rows_v.at[sl],
                         out_hbm.at[base + i].at[:, pl.ds(0, EMB)], ssems[sl])

    def wait_scatter(sl):
        pltpu.make_async_copy(rows_v.at[sl],
                              out_hbm.at[0].at[:, pl.ds(0, EMB)],
                              ssems[sl]).wait()

    start_idx(0, 0)
    start_idx(1, 1)
    start_idx(2, 2)
    wait_idx(0)
    cidx_compute(0)
    start_posseg(0)
    wait_idx(1)
    cidx_compute(1)
    start_posseg(1)
    wait_posseg(0)
    start_tokadd(0)

    def macro_body(m, _):
        i0 = m * SLOTS
        for u in range(SLOTS):
            i = i0 + u
            sl = u
            sl1 = (u + 1) % SLOTS
            sl2 = (u + 2) % SLOTS
            sl3 = (u + 3) % SLOTS

            @pl.when(i + 3 < NSEQ)
            def _():
                start_idx(i + 3, sl3)

            @pl.when(i + 2 < NSEQ)
            def _():
                @pl.when(i >= 2)
                def _():
                    wait_scatter(sl2)
                wait_idx(sl2)
                cidx_compute(sl2)
                start_posseg(sl2)

            @pl.when(i + 1 < NSEQ)
            def _():
                wait_posseg(sl1)
                start_tokadd(sl1)

            wait_tokadd(sl)
            start_scatter(i, sl)
        return 0

    lax.fori_loop(0, NSEQ // SLOTS, macro_body, 0)
    for sl in range(SLOTS):
        wait_scatter(sl)


def kernel(sequence, segment_label, token_table, segment_table,
           position_table, pos_inp):
    seq = jnp.pad(jnp.asarray(sequence, jnp.int32).reshape(2 * BATCH, HALF),
                  ((0, 0), (0, 128 - HALF)))
    lbl = jnp.pad(jnp.asarray(segment_label, jnp.int32).reshape(2 * BATCH, HALF),
                  ((0, 0), (0, 128 - HALF)))
    pidx = jnp.asarray(pos_inp, jnp.int32).reshape(2, HALF)

    run = pl.kernel(
        _body,
        out_type=jax.ShapeDtypeStruct((BATCH, MAX_LEN, 2 * EMB), jnp.float32),
        mesh=plsc.VectorSubcoreMesh(core_axis_name="c", subcore_axis_name="s"),
        compiler_params=pltpu.CompilerParams(use_tc_tiling_on_sc=False),
        scratch_types=[
            pltpu.VMEM((MAX_LEN, EMB), jnp.float32),
            pltpu.VMEM((N_SEG, EMB), jnp.float32),
            pltpu.VMEM((2, HALF), jnp.int32),
            pltpu.VMEM((SLOTS, 2, 128), jnp.int32),
            pltpu.VMEM((SLOTS, 2, 128), jnp.int32),
            pltpu.VMEM((SLOTS, 2, HALF), jnp.int32),
            pltpu.VMEM((SLOTS, 2, HALF), jnp.int32),
            pltpu.VMEM((SLOTS, MAX_LEN, EMB), jnp.float32),
            pltpu.VMEM_SHARED((N_SEG * MAX_LEN, EMB), jnp.float32),
            pltpu.SemaphoreType.DMA,
            [pltpu.SemaphoreType.DMA] * SLOTS,
            [pltpu.SemaphoreType.DMA] * SLOTS,
            [pltpu.SemaphoreType.DMA] * SLOTS,
            [pltpu.SemaphoreType.DMA] * SLOTS,
        ],
    )
    out = run(seq, lbl, token_table, segment_table, position_table, pidx)
    return out[:, :, :EMB]


# speedup vs baseline: 1.1926x; 1.0004x over previous
import functools

import jax
import jax.numpy as jnp
from jax import lax
from jax.experimental import pallas as pl
from jax.experimental.pallas import tpu as pltpu
from jax.experimental.pallas import tpu_sc as plsc

VOCAB = 100000
N_SEG = 3
MAX_LEN = 200
EMB = 64
BATCH = 4096

NC = 2
NS = 16
NW = NC * NS
NSEQ = BATCH // NW
HALF = MAX_LEN // 2
NV = EMB // 16
SLOTS = 4
OFFS = (0, 16, 32, 48, 64, 80, 84)


def _body(seq_hbm, lbl_hbm, tok_hbm, seg_hbm, pos_hbm, pidx_hbm, out_hbm,
          pos_v, seg_v, pidx_v, idx_v, lbl_v, cidx_v, idx2_v, rows_v,
          posseg_sh, bsem, isems, psems, gsems, ssems):
    sid = lax.axis_index("s")
    wid = sid * NC + lax.axis_index("c")
    base = wid * NSEQ

    @pl.when(sid == 0)
    def _():
        pltpu.sync_copy(pidx_hbm, pidx_v)
        pltpu.sync_copy(seg_hbm, seg_v)
        for h in range(2):
            pltpu.async_copy(pos_hbm.at[pidx_v.at[h]],
                             pos_v.at[pl.ds(h * HALF, HALF)], bsem).wait()
        for s in range(N_SEG):
            seg_vals = [seg_v[s, pl.ds(j * 16, 16)] for j in range(NV)]

            def seg_body(l, _, s=s, seg_vals=seg_vals):
                for j in range(NV):
                    rows_v[s, l, pl.ds(j * 16, 16)] = (
                        pos_v[l, pl.ds(j * 16, 16)] + seg_vals[j])
                return 0

            lax.fori_loop(0, MAX_LEN, seg_body, 0)
            pltpu.sync_copy(rows_v.at[s],
                            posseg_sh.at[pl.ds(s * MAX_LEN, MAX_LEN)])
    plsc.subcore_barrier()

    def start_idx(i, sl):
        for h in range(2):
            pltpu.async_copy(seq_hbm.at[2 * (base + i) + h],
                             idx_v.at[sl].at[h], isems[sl])
            pltpu.async_copy(lbl_hbm.at[2 * (base + i) + h],
                             lbl_v.at[sl].at[h], isems[sl])

    def wait_idx(sl):
        for h in range(2):
            pltpu.make_async_copy(seq_hbm.at[0], idx_v.at[sl].at[h],
                                  isems[sl]).wait()
            pltpu.make_async_copy(lbl_hbm.at[0], lbl_v.at[sl].at[h],
                                  isems[sl]).wait()

    def cidx_compute(sl):
        for h in range(2):
            for off in OFFS:
                lv = lax.iota(jnp.int32, 16) + (h * HALF + off)
                sv = lbl_v[sl, h, pl.ds(off, 16)]
                cidx_v[sl, h, pl.ds(off, 16)] = sv * MAX_LEN + lv
                idx2_v[sl, h, pl.ds(off, 16)] = idx_v[sl, h, pl.ds(off, 16)]

    def start_posseg(sl):
        for h in range(2):
            pltpu.async_copy(posseg_sh.at[cidx_v.at[sl].at[h]],
                             rows_v.at[sl].at[pl.ds(h * HALF, HALF)],
                             psems[sl])

    def wait_posseg(sl):
        for h in range(2):
            pltpu.make_async_copy(posseg_sh.at[cidx_v.at[sl].at[h]],
                                  rows_v.at[sl].at[pl.ds(h * HALF, HALF)],
                                  psems[sl]).wait()

    def start_tokadd(sl):
        for h in range(2):
            pltpu.async_copy(tok_hbm.at[idx2_v.at[sl].at[h]],
                             rows_v.at[sl].at[pl.ds(h * HALF, HALF)],
                             gsems[sl], add=True)

    def wait_tokadd(sl):
        for h in range(2):
            pltpu.make_async_copy(tok_hbm.at[idx2_v.at[sl].at[h]],
                                  rows_v.at[sl].at[pl.ds(h * HALF, HALF)],
                                  gsems[sl]).wait()

    def start_scatter(i, sl):
        pltpu.async_copy(rows_v.at[sl],
                         out_hbm.at[base + i].at[:, pl.ds(0, EMB)], ssems[sl])

    def wait_scatter(sl):
        pltpu.make_async_copy(rows_v.at[sl],
                              out_hbm.at[0].at[:, pl.ds(0, EMB)],
                              ssems[sl]).wait()

    start_idx(0, 0)
    start_idx(1, 1)
    start_idx(2, 2)
    wait_idx(0)
    cidx_compute(0)
    start_posseg(0)
    wait_idx(1)
    cidx_compute(1)
    start_posseg(1)
    wait_posseg(0)
    start_tokadd(0)

    def macro_body(m, _):
        i0 = m * SLOTS
        for u in range(SLOTS):
            i = i0 + u
            sl = u
            sl1 = (u + 1) % SLOTS
            sl2 = (u + 2) % SLOTS
            sl3 = (u + 3) % SLOTS

            @pl.when(i + 3 < NSEQ)
            def _():
                start_idx(i + 3, sl3)

            @pl.when(i + 2 < NSEQ)
            def _():
                @pl.when(i >= SLOTS - 2)
                def _():
                    wait_scatter(sl2)
                wait_idx(sl2)
                cidx_compute(sl2)
                start_posseg(sl2)

            @pl.when(i + 1 < NSEQ)
            def _():
                wait_posseg(sl1)
                start_tokadd(sl1)

            wait_tokadd(sl)
            start_scatter(i, sl)
        return 0

    lax.fori_loop(0, NSEQ // SLOTS, macro_body, 0)
    for sl in range(SLOTS):
        wait_scatter(sl)


def kernel(sequence, segment_label, token_table, segment_table,
           position_table, pos_inp):
    seq = jnp.pad(jnp.asarray(sequence, jnp.int32).reshape(2 * BATCH, HALF),
                  ((0, 0), (0, 128 - HALF)))
    lbl = jnp.pad(jnp.asarray(segment_label, jnp.int32).reshape(2 * BATCH, HALF),
                  ((0, 0), (0, 128 - HALF)))
    pidx = jnp.asarray(pos_inp, jnp.int32).reshape(2, HALF)

    run = pl.kernel(
        _body,
        out_type=jax.ShapeDtypeStruct((BATCH, MAX_LEN, 2 * EMB), jnp.float32),
        mesh=plsc.VectorSubcoreMesh(core_axis_name="c", subcore_axis_name="s"),
        compiler_params=pltpu.CompilerParams(use_tc_tiling_on_sc=False),
        scratch_types=[
            pltpu.VMEM((MAX_LEN, EMB), jnp.float32),
            pltpu.VMEM((N_SEG, EMB), jnp.float32),
            pltpu.VMEM((2, HALF), jnp.int32),
            pltpu.VMEM((SLOTS, 2, 128), jnp.int32),
            pltpu.VMEM((SLOTS, 2, 128), jnp.int32),
            pltpu.VMEM((SLOTS, 2, HALF), jnp.int32),
            pltpu.VMEM((SLOTS, 2, HALF), jnp.int32),
            pltpu.VMEM((SLOTS, MAX_LEN, EMB), jnp.float32),
            pltpu.VMEM_SHARED((N_SEG * MAX_LEN, EMB), jnp.float32),
            pltpu.SemaphoreType.DMA,
            [pltpu.SemaphoreType.DMA] * SLOTS,
            [pltpu.SemaphoreType.DMA] * SLOTS,
            [pltpu.SemaphoreType.DMA] * SLOTS,
            [pltpu.SemaphoreType.DMA] * SLOTS,
        ],
    )
    out = run(seq, lbl, token_table, segment_table, position_table, pidx)
    return out[:, :, :EMB]
